# Initial kernel scaffold; baseline (speedup 1.0000x reference)
import jax
import jax.numpy as jnp
from jax.experimental import pallas as pl


def kernel(x_fix, x_varlen, W_fix, W_var):
    raise NotImplementedError("write your pallas kernel here")


# speedup vs baseline: 41.9606x; 41.9606x over previous
import functools

import jax
import jax.numpy as jnp
from jax import lax
from jax.experimental import pallas as pl
from jax.experimental.pallas import tpu as pltpu
from jax.experimental.pallas import tpu_sc as plsc

B = 16384
N_FIX = 26
N_VAR = 4
VOCAB = 100000
L = 50
D = 32

NC = 2
NS = 16
NW = NC * NS

ROWS_PER_BLK = 8
BLKS = B // ROWS_PER_BLK
BLKS_PER_W = BLKS // NW
PAIRS = ROWS_PER_BLK * N_VAR
VAR_G = 16
VAR_GN = ROWS_PER_BLK * N_VAR * L // VAR_G


def _sc_body(wfix_hbm, wvar_hbm, fixidx_hbm, varidx_hbm, out_hbm,
             fixidx_v, varidx_v, fix_buf, var_buf, means, gsem):
  wid = lax.axis_index("s") * NC + lax.axis_index("c")

  def block(i, carry):
    blk = wid * BLKS_PER_W + i
    b0 = blk * ROWS_PER_BLK
    pltpu.sync_copy(fixidx_hbm.at[blk], fixidx_v)
    pltpu.sync_copy(varidx_hbm.at[blk], varidx_v)

    fix_copies = []
    for r in range(ROWS_PER_BLK):
      fix_copies.append(
          pltpu.async_copy(wfix_hbm.at[fixidx_v.at[r]], fix_buf.at[r], gsem))
    var_copies = []
    for g in range(VAR_G):
      var_copies.append(
          pltpu.async_copy(wvar_hbm.at[varidx_v.at[g]], var_buf.at[g], gsem))
    for c in fix_copies:
      c.wait()
    for c in var_copies:
      c.wait()

    inv_l = jnp.float32(1.0 / L)
    for p0 in range(0, PAIRS, 8):
      def red(r, acc):
        new = []
        for k in range(8):
          p = p0 + k
          g = (p * L) // VAR_GN
          off = (p * L) % VAR_GN
          lo = acc[2 * k] + var_buf[g, off + r, pl.ds(0, 16)]
          hi = acc[2 * k + 1] + var_buf[g, off + r, pl.ds(16, 16)]
          new.extend([lo, hi])
        return tuple(new)

      acc0 = tuple(jnp.zeros((16,), jnp.float32) for _ in range(16))
      acc = lax.fori_loop(0, L, red, acc0)
      for k in range(8):
        p = p0 + k
        b, f = p // N_VAR, p % N_VAR
        means[b, f, pl.ds(0, 16)] = acc[2 * k] * inv_l
        means[b, f, pl.ds(16, 16)] = acc[2 * k + 1] * inv_l

    pltpu.sync_copy(fix_buf, out_hbm.at[pl.ds(b0, ROWS_PER_BLK),
                                        pl.ds(0, N_FIX)])
    pltpu.sync_copy(means, out_hbm.at[pl.ds(b0, ROWS_PER_BLK),
                                      pl.ds(N_FIX, N_VAR)])
    return carry

  lax.fori_loop(0, BLKS_PER_W, block, 0)


@jax.jit
def kernel(x_fix, x_varlen, W_fix, W_var):
  wfix = W_fix.reshape(N_FIX * VOCAB, D)
  wvar = W_var.reshape(N_VAR * VOCAB, D)
  fix_gidx = (x_fix.astype(jnp.int32)
              + (jnp.arange(N_FIX, dtype=jnp.int32) * VOCAB)[None, :])
  var_gidx = (x_varlen.astype(jnp.int32)
              + (jnp.arange(N_VAR, dtype=jnp.int32) * VOCAB)[None, :, None])
  fix_gidx = fix_gidx.reshape(BLKS, ROWS_PER_BLK, N_FIX)
  var_gidx = var_gidx.reshape(BLKS, VAR_G, VAR_GN)

  mesh = plsc.VectorSubcoreMesh(core_axis_name="c", subcore_axis_name="s")
  out = pl.kernel(
      _sc_body,
      out_type=jax.ShapeDtypeStruct((B, N_FIX + N_VAR, D), jnp.float32),
      mesh=mesh,
      compiler_params=pltpu.CompilerParams(use_tc_tiling_on_sc=False),
      scratch_types=[
          pltpu.VMEM((ROWS_PER_BLK, N_FIX), jnp.int32),
          pltpu.VMEM((VAR_G, VAR_GN), jnp.int32),
          pltpu.VMEM((ROWS_PER_BLK, N_FIX, D), jnp.float32),
          pltpu.VMEM((VAR_G, VAR_GN, D), jnp.float32),
          pltpu.VMEM((ROWS_PER_BLK, N_VAR, D), jnp.float32),
          pltpu.SemaphoreType.DMA,
      ],
  )(wfix, wvar, fix_gidx, var_gidx)
  return out.reshape(B, (N_FIX + N_VAR) * D)
